# baseline (speedup 1.0000x reference)
import jax
import jax.numpy as jnp
from jax.experimental import pallas as pl
from jax.experimental.pallas import tpu as pltpu

NUM_EMBEDDINGS = 1024
EMBEDDING_DIM = 64
COMMITMENT_COST = 0.25

N_BATCH = 32
N_POS = 1024
N_ROWS = N_BATCH * N_POS
BATCH_PER_STEP = 8


def _vq_body(z_ref, cb_ref, zq_ref, loss_ref, perp_ref, acc_ref, counts_ref):
    step = pl.program_id(0)
    nsteps = pl.num_programs(0)

    cb = cb_ref[...]
    s2 = jnp.sum(cb * cb, axis=1)[:, None]
    cb2 = 2.0 * cb
    cb16 = cb.astype(jnp.bfloat16)
    fiota = None

    @pl.when(step == 0)
    def _init():
        counts_ref[...] = jnp.zeros_like(counts_ref)
        acc_ref[0, 0] = 0.0

    for u in range(BATCH_PER_STEP):
        zb = z_ref[u]

        s1 = jnp.sum(zb * zb, axis=0, keepdims=True)
        t = jax.lax.dot_general(cb2, zb, (((1,), (0,)), ((), ())),
                                preferred_element_type=jnp.float32)
        d = (s1 + s2) - t

        m = jnp.min(d, axis=0, keepdims=True)
        if fiota is None:
            fiota = jax.lax.broadcasted_iota(
                jnp.int32, d.shape, 0).astype(jnp.float32)
        fidx = jnp.min(jnp.where(d == m, fiota, jnp.float32(NUM_EMBEDDINGS)),
                       axis=0, keepdims=True)

        onehot = fiota == fidx
        enc = onehot.astype(jnp.bfloat16)
        zq = jax.lax.dot_general(cb16, enc, (((0,), (0,)), ((), ())),
                                 preferred_element_type=jnp.float32)

        zq_ref[u] = zb + (zq - zb)

        err = zq - zb
        of = jnp.where(onehot, 1.0, 0.0)
        part = of[:, 0:128]
        for g in range(1, 8):
            part = part + of[:, g * 128:(g + 1) * 128]
        counts_ref[...] += part
        acc_ref[0, 0] += jnp.sum(err * err)

    @pl.when(step == nsteps - 1)
    def _fini():
        mse = acc_ref[0, 0] / jnp.float32(N_ROWS * EMBEDDING_DIM)
        loss_ref[...] = jnp.reshape(mse + COMMITMENT_COST * mse, (1, 1))
        counts = jnp.sum(counts_ref[...], axis=1, keepdims=True)
        p = counts / jnp.float32(N_ROWS)
        ent = -jnp.sum(p * jnp.log(p + 1e-10))
        perp_ref[...] = jnp.reshape(jnp.exp(ent), (1, 1))


def kernel(z_e, codebook):
    zr = z_e.reshape(N_BATCH, EMBEDDING_DIM, N_POS)

    zq, loss, perp = pl.pallas_call(
        _vq_body,
        grid=(N_BATCH // BATCH_PER_STEP,),
        in_specs=[
            pl.BlockSpec((BATCH_PER_STEP, EMBEDDING_DIM, N_POS),
                         lambda i: (i, 0, 0)),
            pl.BlockSpec((NUM_EMBEDDINGS, EMBEDDING_DIM), lambda i: (0, 0)),
        ],
        out_specs=[
            pl.BlockSpec((BATCH_PER_STEP, EMBEDDING_DIM, N_POS),
                         lambda i: (i, 0, 0)),
            pl.BlockSpec((1, 1), lambda i: (0, 0)),
            pl.BlockSpec((1, 1), lambda i: (0, 0)),
        ],
        out_shape=[
            jax.ShapeDtypeStruct((N_BATCH, EMBEDDING_DIM, N_POS), jnp.float32),
            jax.ShapeDtypeStruct((1, 1), jnp.float32),
            jax.ShapeDtypeStruct((1, 1), jnp.float32),
        ],
        scratch_shapes=[
            pltpu.SMEM((1, 1), jnp.float32),
            pltpu.VMEM((NUM_EMBEDDINGS, 128), jnp.float32),
        ],
    )(zr, codebook)

    z_q_out = zq.reshape(z_e.shape)
    return (z_q_out, loss[0, 0], perp[0, 0], 0)


# speedup vs baseline: 1.0116x; 1.0116x over previous
import jax
import jax.numpy as jnp
from jax.experimental import pallas as pl
from jax.experimental.pallas import tpu as pltpu

NUM_EMBEDDINGS = 1024
EMBEDDING_DIM = 64
COMMITMENT_COST = 0.25

N_BATCH = 32
N_POS = 1024
N_ROWS = N_BATCH * N_POS
BATCH_PER_STEP = 8


def _vq_body(z_ref, cb_ref, zq_ref, loss_ref, perp_ref, acc_ref, counts_ref):
    step = pl.program_id(0)
    nsteps = pl.num_programs(0)

    cb = cb_ref[...]
    s2 = jnp.sum(cb * cb, axis=1)[:, None]
    cb2 = 2.0 * cb
    cb16 = cb.astype(jnp.bfloat16)
    fiota = None

    @pl.when(step == 0)
    def _init():
        counts_ref[...] = jnp.zeros_like(counts_ref)
        acc_ref[0, 0] = 0.0

    for u in range(BATCH_PER_STEP):
        zb = z_ref[u]

        s1 = jnp.sum(zb * zb, axis=0, keepdims=True)
        t = jax.lax.dot_general(cb2, zb, (((1,), (0,)), ((), ())),
                                preferred_element_type=jnp.float32)
        d = (s1 + s2) - t

        m = jnp.min(d, axis=0, keepdims=True)
        if fiota is None:
            fiota = jax.lax.broadcasted_iota(
                jnp.int32, d.shape, 0).astype(jnp.float32)
        fidx = jnp.min(jnp.where(d == m, fiota, jnp.float32(NUM_EMBEDDINGS)),
                       axis=0, keepdims=True)

        onehot = fiota == fidx
        enc = onehot.astype(jnp.bfloat16)
        zq = jax.lax.dot_general(cb16, enc, (((0,), (0,)), ((), ())),
                                 preferred_element_type=jnp.float32)

        zq_ref[u] = zb + (zq - zb)

        err = zq - zb
        counts_ref[...] += jnp.sum(
            jnp.where(onehot, 1.0, 0.0), axis=1, keepdims=True)
        acc_ref[0, 0] += jnp.sum(err * err)

    @pl.when(step == nsteps - 1)
    def _fini():
        mse = acc_ref[0, 0] / jnp.float32(N_ROWS * EMBEDDING_DIM)
        loss_ref[...] = jnp.reshape(mse + COMMITMENT_COST * mse, (1, 1))
        p = counts_ref[...] / jnp.float32(N_ROWS)
        ent = -jnp.sum(p * jnp.log(p + 1e-10))
        perp_ref[...] = jnp.reshape(jnp.exp(ent), (1, 1))


def kernel(z_e, codebook):
    zr = z_e.reshape(N_BATCH, EMBEDDING_DIM, N_POS)

    zq, loss, perp = pl.pallas_call(
        _vq_body,
        grid=(N_BATCH // BATCH_PER_STEP,),
        in_specs=[
            pl.BlockSpec((BATCH_PER_STEP, EMBEDDING_DIM, N_POS),
                         lambda i: (i, 0, 0)),
            pl.BlockSpec((NUM_EMBEDDINGS, EMBEDDING_DIM), lambda i: (0, 0)),
        ],
        out_specs=[
            pl.BlockSpec((BATCH_PER_STEP, EMBEDDING_DIM, N_POS),
                         lambda i: (i, 0, 0)),
            pl.BlockSpec((1, 1), lambda i: (0, 0)),
            pl.BlockSpec((1, 1), lambda i: (0, 0)),
        ],
        out_shape=[
            jax.ShapeDtypeStruct((N_BATCH, EMBEDDING_DIM, N_POS), jnp.float32),
            jax.ShapeDtypeStruct((1, 1), jnp.float32),
            jax.ShapeDtypeStruct((1, 1), jnp.float32),
        ],
        scratch_shapes=[
            pltpu.SMEM((1, 1), jnp.float32),
            pltpu.VMEM((NUM_EMBEDDINGS, 1), jnp.float32),
        ],
    )(zr, codebook)

    z_q_out = zq.reshape(z_e.shape)
    return (z_q_out, loss[0, 0], perp[0, 0], 0)
